# baseline (speedup 1.0000x reference)
import functools

import jax
import jax.numpy as jnp
import numpy as np
from jax import lax
from jax.experimental import pallas as pl
from jax.experimental.pallas import tpu as pltpu
from jax.experimental.pallas import tpu_sc as plsc

_B, _C, _H, _W = 8, 96, 32, 32
_N = _H * _W
_K = 9
_GROUPS = 4
_GSIZE = _C // _GROUPS

_NW = 32
_NODES_PW = _B * _N // _NW
_SUB = 32
_FLAT_SUB = _SUB * _K
_IDX_ROWS = _FLAT_SUB // 96
_CV = _C // 16


def _sim_topk_body(x_nc_ref, x_cn_ref, W1_ref, xt_ref, idx_ref, val_ref,
                   sim_ref):
    b = pl.program_id(0)
    x_nc = x_nc_ref[0]
    x_cn = x_cn_ref[0]

    rs = jnp.sum(x_nc * x_nc, axis=1, keepdims=True)
    inv_r = 1.0 / jnp.maximum(jnp.sqrt(rs), 1e-12)
    xn = x_nc * inv_r
    cs = jnp.sum(x_cn * x_cn, axis=0, keepdims=True)
    inv_c = 1.0 / jnp.maximum(jnp.sqrt(cs), 1e-12)
    xnT = x_cn * inv_c

    sim_ref[...] = jnp.dot(xn, xnT, preferred_element_type=jnp.float32)

    lane16 = jax.lax.broadcasted_iota(jnp.int32, (_N, 16), 1)
    iota_col = jax.lax.broadcasted_iota(
        jnp.int32, (_N, 8), 0).astype(jnp.float32)
    idx_buf = jnp.zeros((_N, 16), jnp.int32)
    val_buf = jnp.zeros((_N, 16), jnp.float32)
    deg = jnp.zeros((_N, 1), jnp.float32)
    for k in range(_K):
        s = sim_ref[...]
        m = jnp.max(s, axis=1, keepdims=True)
        eq = s == m
        onehot = eq.astype(jnp.float32)
        sel_f = jnp.dot(onehot, iota_col,
                        preferred_element_type=jnp.float32,
                        precision=jax.lax.Precision.HIGHEST)[:, :1]
        sel = jnp.minimum((sel_f + 0.5).astype(jnp.int32), _N - 1)
        sim_ref[...] = jnp.where(eq, -3.0, s)
        idx_buf = jnp.where(lane16 == k, sel, idx_buf)
        val_buf = jnp.where(lane16 == k, m, val_buf)
        deg = deg + m
    idx_ref[0] = idx_buf + b * _N
    val_ref[0] = val_buf * (1.0 / (deg + 1e-6))
    xt_ref[0] = jnp.dot(xn, W1_ref[...], preferred_element_type=jnp.float32)


def _gn_lin_body(agg_ref, bias_ref, gnw_ref, gnb_ref, M_ref, Wt_ref, out_ref):
    a = agg_ref[0] + bias_ref[...]
    csum = jnp.sum(a, axis=0, keepdims=True)
    csq = jnp.sum(a * a, axis=0, keepdims=True)
    M = M_ref[...]
    mean = jnp.dot(csum, M, preferred_element_type=jnp.float32)
    ex2 = jnp.dot(csq, M, preferred_element_type=jnp.float32)
    var = ex2 - mean * mean
    hn = (a - mean) * jax.lax.rsqrt(var + 1e-5)
    hn = hn * gnw_ref[...] + gnb_ref[...]
    h = hn * (1.0 / (1.0 + jnp.exp(-hn)))
    out_ref[0] = jnp.dot(h, Wt_ref[...], preferred_element_type=jnp.float32)


def _sc_agg_body(table_ref, gidx_ref, valn_ref, out_ref,
                 idx_v, val_v, rows_v0, rows_v1, out_v, sem0, sem1):
    wid = lax.axis_index("s") * 2 + lax.axis_index("c")
    base = wid * _NODES_PW
    idx_rows_pw = _NODES_PW * _K // 96
    n_sub = _NODES_PW // _SUB

    pltpu.sync_copy(gidx_ref.at[pl.ds(wid * idx_rows_pw, idx_rows_pw)], idx_v)
    pltpu.sync_copy(valn_ref.at[pl.ds(base * 16, _NODES_PW * 16)], val_v)

    rows = [rows_v0, rows_v1]
    sems = [sem0, sem1]

    def start(j):
        buf, sem = rows[j % 2], sems[j % 2]
        return [
            pltpu.async_copy(table_ref.at[idx_v.at[j * _IDX_ROWS + i]],
                             buf.at[pl.ds(i * 96, 96)], sem)
            for i in range(_IDX_ROWS)
        ]

    pending = {0: start(0)}
    for j in range(n_sub):
        if j + 1 < n_sub:
            pending[j + 1] = start(j + 1)
        for cp in pending.pop(j):
            cp.wait()
        buf = rows[j % 2]

        def node(n, carry):
            w16 = val_v[pl.ds((j * _SUB + n) * 16, 16)]
            accs = [jnp.zeros((16,), jnp.float32) for _ in range(_CV)]
            for k in range(_K):
                r = n * _K + k
                w = jnp.broadcast_to(w16[k], (16,))
                for c in range(_CV):
                    accs[c] = accs[c] + w * buf[r, pl.ds(c * 16, 16)]
            for c in range(_CV):
                out_v[n, pl.ds(c * 16, 16)] = accs[c]
            return carry

        lax.fori_loop(0, _SUB, node, 0)
        pltpu.sync_copy(out_v, out_ref.at[pl.ds(base + j * _SUB, _SUB)])


def _sc_aggregate(table, gidx2d, valn_flat):
    mesh = plsc.VectorSubcoreMesh(core_axis_name="c", subcore_axis_name="s")
    f = functools.partial(
        pl.kernel,
        out_type=jax.ShapeDtypeStruct((_B * _N, _C), jnp.float32),
        mesh=mesh,
        scratch_types=[
            pltpu.VMEM((_NODES_PW * _K // 96, 96), jnp.int32),
            pltpu.VMEM((_NODES_PW * 16,), jnp.float32),
            pltpu.VMEM((_FLAT_SUB, 128), jnp.float32),
            pltpu.VMEM((_FLAT_SUB, 128), jnp.float32),
            pltpu.VMEM((_SUB, _C), jnp.float32),
            pltpu.SemaphoreType.DMA,
            pltpu.SemaphoreType.DMA,
        ],
    )(_sc_agg_body)
    return f(table, gidx2d, valn_flat)


@jax.jit
def _run(x, W1, b1, W2, b2, gn1w, gn1b, gn2w, gn2b):
    x_cn = x.reshape(_B, _C, _N)
    x_nc = x_cn.transpose(0, 2, 1)
    g = np.arange(_C) // _GSIZE
    M = jnp.asarray((g[:, None] == g[None, :]).astype(np.float32)
                    / (_N * _GSIZE))

    full = lambda *shape: pl.BlockSpec(shape, lambda b: (0,) * len(shape))
    batch3 = lambda d1, d2: pl.BlockSpec((1, d1, d2), lambda b: (b, 0, 0))

    xt1, idx, val = pl.pallas_call(
        _sim_topk_body,
        grid=(_B,),
        in_specs=[batch3(_N, _C), batch3(_C, _N), full(_C, 128)],
        out_specs=[batch3(_N, 128), batch3(_N, 16), batch3(_N, 16)],
        out_shape=[jax.ShapeDtypeStruct((_B, _N, 128), jnp.float32),
                   jax.ShapeDtypeStruct((_B, _N, 16), jnp.int32),
                   jax.ShapeDtypeStruct((_B, _N, 16), jnp.float32)],
        scratch_shapes=[pltpu.VMEM((_N, _N), jnp.float32)],
    )(x_nc, x_cn, jnp.pad(W1, ((0, 0), (0, 128 - _C))))

    gidx2d = idx[:, :, :_K].reshape(_B * _N * _K // 96, 96)
    valn = val.reshape(_B * _N * 16)

    def gn_lin(agg_flat, bias, gnw, gnb, Wt):
        return pl.pallas_call(
            _gn_lin_body,
            grid=(_B,),
            in_specs=[batch3(_N, _C), full(1, _C), full(1, _C), full(1, _C),
                      full(_C, _C), full(_C, 128)],
            out_specs=batch3(_N, 128),
            out_shape=jax.ShapeDtypeStruct((_B, _N, 128), jnp.float32),
        )(agg_flat.reshape(_B, _N, _C), bias.reshape(1, _C),
          gnw.reshape(1, _C), gnb.reshape(1, _C), M, Wt)

    agg1 = _sc_aggregate(xt1.reshape(_B * _N, 128), gidx2d, valn)
    xt2 = gn_lin(agg1, b1, gn1w, gn1b, jnp.pad(W2, ((0, 0), (0, 128 - _C))))
    agg2 = _sc_aggregate(xt2.reshape(_B * _N, 128), gidx2d, valn)
    out = gn_lin(agg2, b2, gn2w, gn2b, jnp.eye(_C, 128, dtype=jnp.float32))
    return out[:, :, :_C].transpose(0, 2, 1).reshape(_B, _C, _H, _W)


def kernel(x, W1, b1, W2, b2, gn1_w, gn1_b, gn2_w, gn2_b):
    return _run(x, W1, b1, W2, b2, gn1_w, gn1_b, gn2_w, gn2_b)


# speedup vs baseline: 3.3942x; 3.3942x over previous
import functools

import jax
import jax.numpy as jnp
import numpy as np
from jax import lax
from jax.experimental import pallas as pl
from jax.experimental.pallas import tpu as pltpu
from jax.experimental.pallas import tpu_sc as plsc

_B, _C, _H, _W = 8, 96, 32, 32
_N = _H * _W
_K = 9
_GROUPS = 4
_GSIZE = _C // _GROUPS

_NW = 32
_NODES_PW = _B * _N // _NW
_SUB = 32
_FLAT_SUB = _SUB * _K
_IDX_ROWS = _FLAT_SUB // 96
_CV = _C // 16


def _sim_topk_body(x_nc_ref, x_cn_ref, W1_ref, xt_ref, idx_ref, val_ref,
                   sim_ref):
    b = pl.program_id(0)
    x_nc = x_nc_ref[0]
    x_cn = x_cn_ref[0]

    rs = jnp.sum(x_nc * x_nc, axis=1, keepdims=True)
    inv_r = 1.0 / jnp.maximum(jnp.sqrt(rs), 1e-12)
    xn = x_nc * inv_r
    cs = jnp.sum(x_cn * x_cn, axis=0, keepdims=True)
    inv_c = 1.0 / jnp.maximum(jnp.sqrt(cs), 1e-12)
    xnT = x_cn * inv_c

    sim_ref[...] = jnp.dot(xn, xnT, preferred_element_type=jnp.float32)

    lane16 = jax.lax.broadcasted_iota(jnp.int32, (_N, 16), 1)
    lane = jax.lax.broadcasted_iota(jnp.int32, (_N, _N), 1)
    s_bits = jax.lax.bitcast_convert_type(sim_ref[...], jnp.int32)
    keys = jax.lax.bitcast_convert_type(
        (s_bits & jnp.int32(~1023)) | (jnp.int32(1023) - lane), jnp.float32)
    sim_ref[...] = keys

    idx_buf = jnp.zeros((_N, 16), jnp.int32)
    val_buf = jnp.zeros((_N, 16), jnp.float32)
    deg = jnp.zeros((_N, 1), jnp.float32)
    for k in range(_K):
        s = sim_ref[...]
        m = jnp.max(s, axis=1, keepdims=True)
        sim_ref[...] = jnp.where(s == m, -3.0, s)
        mi = jax.lax.bitcast_convert_type(m, jnp.int32)
        sel = jnp.int32(1023) - (mi & jnp.int32(1023))
        mv = jax.lax.bitcast_convert_type(mi & jnp.int32(~1023), jnp.float32)
        idx_buf = jnp.where(lane16 == k, sel, idx_buf)
        val_buf = jnp.where(lane16 == k, mv, val_buf)
        deg = deg + mv
    idx_ref[0] = idx_buf + b * _N
    val_ref[0] = val_buf * (1.0 / (deg + 1e-6))
    xt_ref[0] = jnp.dot(xn, W1_ref[...], preferred_element_type=jnp.float32)


def _gn_lin_body(agg_ref, bias_ref, gnw_ref, gnb_ref, M_ref, Wt_ref, out_ref):
    a = agg_ref[0] + bias_ref[...]
    csum = jnp.sum(a, axis=0, keepdims=True)
    csq = jnp.sum(a * a, axis=0, keepdims=True)
    M = M_ref[...]
    mean = jnp.dot(csum, M, preferred_element_type=jnp.float32)
    ex2 = jnp.dot(csq, M, preferred_element_type=jnp.float32)
    var = ex2 - mean * mean
    hn = (a - mean) * jax.lax.rsqrt(var + 1e-5)
    hn = hn * gnw_ref[...] + gnb_ref[...]
    h = hn * (1.0 / (1.0 + jnp.exp(-hn)))
    out_ref[0] = jnp.dot(h, Wt_ref[...], preferred_element_type=jnp.float32)


def _sc_agg_body(table_ref, gidx_ref, valn_ref, out_ref,
                 idx_v, val_v, rows_v0, rows_v1, out_v, sem0, sem1):
    wid = lax.axis_index("s") * 2 + lax.axis_index("c")
    base = wid * _NODES_PW
    idx_rows_pw = _NODES_PW * _K // 96
    n_sub = _NODES_PW // _SUB

    pltpu.sync_copy(gidx_ref.at[pl.ds(wid * idx_rows_pw, idx_rows_pw)], idx_v)
    pltpu.sync_copy(valn_ref.at[pl.ds(base * 16, _NODES_PW * 16)], val_v)

    rows = [rows_v0, rows_v1]
    sems = [sem0, sem1]

    def start(j):
        buf, sem = rows[j % 2], sems[j % 2]
        return [
            pltpu.async_copy(table_ref.at[idx_v.at[j * _IDX_ROWS + i]],
                             buf.at[pl.ds(i * 96, 96)], sem)
            for i in range(_IDX_ROWS)
        ]

    pending = {0: start(0)}
    for j in range(n_sub):
        if j + 1 < n_sub:
            pending[j + 1] = start(j + 1)
        for cp in pending.pop(j):
            cp.wait()
        buf = rows[j % 2]

        def node(n, carry):
            w16 = val_v[pl.ds((j * _SUB + n) * 16, 16)]
            accs = [jnp.zeros((16,), jnp.float32) for _ in range(_CV)]
            for k in range(_K):
                r = n * _K + k
                w = jnp.broadcast_to(w16[k], (16,))
                for c in range(_CV):
                    accs[c] = accs[c] + w * buf[r, pl.ds(c * 16, 16)]
            for c in range(_CV):
                out_v[n, pl.ds(c * 16, 16)] = accs[c]
            return carry

        lax.fori_loop(0, _SUB, node, 0)
        pltpu.sync_copy(out_v, out_ref.at[pl.ds(base + j * _SUB, _SUB)])


def _sc_aggregate(table, gidx2d, valn_flat):
    mesh = plsc.VectorSubcoreMesh(core_axis_name="c", subcore_axis_name="s")
    f = functools.partial(
        pl.kernel,
        out_type=jax.ShapeDtypeStruct((_B * _N, _C), jnp.float32),
        mesh=mesh,
        scratch_types=[
            pltpu.VMEM((_NODES_PW * _K // 96, 96), jnp.int32),
            pltpu.VMEM((_NODES_PW * 16,), jnp.float32),
            pltpu.VMEM((_FLAT_SUB, 128), jnp.float32),
            pltpu.VMEM((_FLAT_SUB, 128), jnp.float32),
            pltpu.VMEM((_SUB, _C), jnp.float32),
            pltpu.SemaphoreType.DMA,
            pltpu.SemaphoreType.DMA,
        ],
    )(_sc_agg_body)
    return f(table, gidx2d, valn_flat)


@jax.jit
def _run(x, W1, b1, W2, b2, gn1w, gn1b, gn2w, gn2b):
    x_cn = x.reshape(_B, _C, _N)
    x_nc = x_cn.transpose(0, 2, 1)
    g = np.arange(_C) // _GSIZE
    M = jnp.asarray((g[:, None] == g[None, :]).astype(np.float32)
                    / (_N * _GSIZE))

    full = lambda *shape: pl.BlockSpec(shape, lambda b: (0,) * len(shape))
    batch3 = lambda d1, d2: pl.BlockSpec((1, d1, d2), lambda b: (b, 0, 0))

    xt1, idx, val = pl.pallas_call(
        _sim_topk_body,
        grid=(_B,),
        in_specs=[batch3(_N, _C), batch3(_C, _N), full(_C, 128)],
        out_specs=[batch3(_N, 128), batch3(_N, 16), batch3(_N, 16)],
        out_shape=[jax.ShapeDtypeStruct((_B, _N, 128), jnp.float32),
                   jax.ShapeDtypeStruct((_B, _N, 16), jnp.int32),
                   jax.ShapeDtypeStruct((_B, _N, 16), jnp.float32)],
        scratch_shapes=[pltpu.VMEM((_N, _N), jnp.float32)],
    )(x_nc, x_cn, jnp.pad(W1, ((0, 0), (0, 128 - _C))))

    gidx2d = idx[:, :, :_K].reshape(_B * _N * _K // 96, 96)
    valn = val.reshape(_B * _N * 16)

    def gn_lin(agg_flat, bias, gnw, gnb, Wt):
        return pl.pallas_call(
            _gn_lin_body,
            grid=(_B,),
            in_specs=[batch3(_N, _C), full(1, _C), full(1, _C), full(1, _C),
                      full(_C, _C), full(_C, 128)],
            out_specs=batch3(_N, 128),
            out_shape=jax.ShapeDtypeStruct((_B, _N, 128), jnp.float32),
        )(agg_flat.reshape(_B, _N, _C), bias.reshape(1, _C),
          gnw.reshape(1, _C), gnb.reshape(1, _C), M, Wt)

    agg1 = _sc_aggregate(xt1.reshape(_B * _N, 128), gidx2d, valn)
    xt2 = gn_lin(agg1, b1, gn1w, gn1b, jnp.pad(W2, ((0, 0), (0, 128 - _C))))
    agg2 = _sc_aggregate(xt2.reshape(_B * _N, 128), gidx2d, valn)
    out = gn_lin(agg2, b2, gn2w, gn2b, jnp.eye(_C, 128, dtype=jnp.float32))
    return out[:, :, :_C].transpose(0, 2, 1).reshape(_B, _C, _H, _W)


def kernel(x, W1, b1, W2, b2, gn1_w, gn1_b, gn2_w, gn2_b):
    return _run(x, W1, b1, W2, b2, gn1_w, gn1_b, gn2_w, gn2_b)


# speedup vs baseline: 3.8222x; 1.1261x over previous
import functools

import jax
import jax.numpy as jnp
import numpy as np
from jax import lax
from jax.experimental import pallas as pl
from jax.experimental.pallas import tpu as pltpu
from jax.experimental.pallas import tpu_sc as plsc

_B, _C, _H, _W = 8, 96, 32, 32
_N = _H * _W
_K = 9
_GROUPS = 4
_GSIZE = _C // _GROUPS

_NW = 32
_NODES_PW = _B * _N // _NW
_SUB = 32
_FLAT_SUB = _SUB * _K
_IDX_ROWS = _FLAT_SUB // 96
_CV = _C // 16


def _sim_topk_body(x_nc_ref, x_cn_ref, W1_ref, agg_ref, idx_ref, val_ref,
                   sim_ref, key_ref):
    b = pl.program_id(0)
    x_nc = x_nc_ref[0]
    x_cn = x_cn_ref[0]

    rs = jnp.sum(x_nc * x_nc, axis=1, keepdims=True)
    inv_r = 1.0 / jnp.maximum(jnp.sqrt(rs), 1e-12)
    xn = x_nc * inv_r
    cs = jnp.sum(x_cn * x_cn, axis=0, keepdims=True)
    inv_c = 1.0 / jnp.maximum(jnp.sqrt(cs), 1e-12)
    xnT = x_cn * inv_c

    sim = jnp.dot(xn, xnT, preferred_element_type=jnp.float32)

    lane16 = jax.lax.broadcasted_iota(jnp.int32, (_N, 16), 1)
    lane = jax.lax.broadcasted_iota(jnp.int32, (_N, _N), 1)
    s_bits = jax.lax.bitcast_convert_type(sim, jnp.int32)
    keys = jax.lax.bitcast_convert_type(
        (s_bits & jnp.int32(~1023)) | (jnp.int32(1023) - lane), jnp.float32)
    key_ref[...] = keys
    sim_ref[...] = keys

    idx_buf = jnp.zeros((_N, 16), jnp.int32)
    val_buf = jnp.zeros((_N, 16), jnp.float32)
    deg = jnp.zeros((_N, 1), jnp.float32)
    m = None
    for k in range(_K):
        s = sim_ref[...]
        m = jnp.max(s, axis=1, keepdims=True)
        sim_ref[...] = jnp.where(s == m, -3.0, s)
        mi = jax.lax.bitcast_convert_type(m, jnp.int32)
        sel = jnp.int32(1023) - (mi & jnp.int32(1023))
        mv = jax.lax.bitcast_convert_type(mi & jnp.int32(~1023), jnp.float32)
        idx_buf = jnp.where(lane16 == k, sel, idx_buf)
        val_buf = jnp.where(lane16 == k, mv, val_buf)
        deg = deg + mv
    inv_deg = 1.0 / (deg + 1e-6)
    idx_ref[0] = idx_buf + b * _N
    val_ref[0] = val_buf * inv_deg

    keys0 = key_ref[...]
    kv = jax.lax.bitcast_convert_type(
        jax.lax.bitcast_convert_type(keys0, jnp.int32) & jnp.int32(~1023),
        jnp.float32)
    A = jnp.where(keys0 >= m, kv * inv_deg, 0.0)
    xt1 = jnp.dot(xn, W1_ref[...], preferred_element_type=jnp.float32)
    agg_ref[0] = jnp.dot(A, xt1, preferred_element_type=jnp.float32)


def _gn_lin_body(agg_ref, bias_ref, gnw_ref, gnb_ref, M_ref, Wt_ref, out_ref):
    a = agg_ref[0] + bias_ref[...]
    csum = jnp.sum(a, axis=0, keepdims=True)
    csq = jnp.sum(a * a, axis=0, keepdims=True)
    M = M_ref[...]
    mean = jnp.dot(csum, M, preferred_element_type=jnp.float32)
    ex2 = jnp.dot(csq, M, preferred_element_type=jnp.float32)
    var = ex2 - mean * mean
    hn = (a - mean) * jax.lax.rsqrt(var + 1e-5)
    hn = hn * gnw_ref[...] + gnb_ref[...]
    h = hn * (1.0 / (1.0 + jnp.exp(-hn)))
    out_ref[0] = jnp.dot(h, Wt_ref[...], preferred_element_type=jnp.float32)


def _sc_agg_body(table_ref, gidx_ref, valn_ref, out_ref,
                 idx_v, val_v, rows_v0, rows_v1, out_v, sem0, sem1):
    wid = lax.axis_index("s") * 2 + lax.axis_index("c")
    base = wid * _NODES_PW
    idx_rows_pw = _NODES_PW * _K // 96
    n_sub = _NODES_PW // _SUB

    pltpu.sync_copy(gidx_ref.at[pl.ds(wid * idx_rows_pw, idx_rows_pw)], idx_v)
    pltpu.sync_copy(valn_ref.at[pl.ds(base * 16, _NODES_PW * 16)], val_v)

    rows = [rows_v0, rows_v1]
    sems = [sem0, sem1]

    def start(j):
        buf, sem = rows[j % 2], sems[j % 2]
        return [
            pltpu.async_copy(table_ref.at[idx_v.at[j * _IDX_ROWS + i]],
                             buf.at[pl.ds(i * 96, 96)], sem)
            for i in range(_IDX_ROWS)
        ]

    pending = {0: start(0)}
    for j in range(n_sub):
        if j + 1 < n_sub:
            pending[j + 1] = start(j + 1)
        for cp in pending.pop(j):
            cp.wait()
        buf = rows[j % 2]

        def node(n, carry):
            w16 = val_v[pl.ds((j * _SUB + n) * 16, 16)]
            accs = [jnp.zeros((16,), jnp.float32) for _ in range(_CV)]
            for k in range(_K):
                r = n * _K + k
                w = jnp.broadcast_to(w16[k], (16,))
                for c in range(_CV):
                    accs[c] = accs[c] + w * buf[r, pl.ds(c * 16, 16)]
            for c in range(_CV):
                out_v[n, pl.ds(c * 16, 16)] = accs[c]
            return carry

        lax.fori_loop(0, _SUB, node, 0)
        pltpu.sync_copy(out_v, out_ref.at[pl.ds(base + j * _SUB, _SUB)])


def _sc_aggregate(table, gidx2d, valn_flat):
    mesh = plsc.VectorSubcoreMesh(core_axis_name="c", subcore_axis_name="s")
    f = functools.partial(
        pl.kernel,
        out_type=jax.ShapeDtypeStruct((_B * _N, _C), jnp.float32),
        mesh=mesh,
        scratch_types=[
            pltpu.VMEM((_NODES_PW * _K // 96, 96), jnp.int32),
            pltpu.VMEM((_NODES_PW * 16,), jnp.float32),
            pltpu.VMEM((_FLAT_SUB, 128), jnp.float32),
            pltpu.VMEM((_FLAT_SUB, 128), jnp.float32),
            pltpu.VMEM((_SUB, _C), jnp.float32),
            pltpu.SemaphoreType.DMA,
            pltpu.SemaphoreType.DMA,
        ],
    )(_sc_agg_body)
    return f(table, gidx2d, valn_flat)


@jax.jit
def _run(x, W1, b1, W2, b2, gn1w, gn1b, gn2w, gn2b):
    x_cn = x.reshape(_B, _C, _N)
    x_nc = x_cn.transpose(0, 2, 1)
    g = np.arange(_C) // _GSIZE
    M = jnp.asarray((g[:, None] == g[None, :]).astype(np.float32)
                    / (_N * _GSIZE))

    full = lambda *shape: pl.BlockSpec(shape, lambda b: (0,) * len(shape))
    batch3 = lambda d1, d2: pl.BlockSpec((1, d1, d2), lambda b: (b, 0, 0))

    agg1, idx, val = pl.pallas_call(
        _sim_topk_body,
        grid=(_B,),
        in_specs=[batch3(_N, _C), batch3(_C, _N), full(_C, _C)],
        out_specs=[batch3(_N, _C), batch3(_N, 16), batch3(_N, 16)],
        out_shape=[jax.ShapeDtypeStruct((_B, _N, _C), jnp.float32),
                   jax.ShapeDtypeStruct((_B, _N, 16), jnp.int32),
                   jax.ShapeDtypeStruct((_B, _N, 16), jnp.float32)],
        scratch_shapes=[pltpu.VMEM((_N, _N), jnp.float32),
                        pltpu.VMEM((_N, _N), jnp.float32)],
    )(x_nc, x_cn, W1)

    gidx2d = idx[:, :, :_K].reshape(_B * _N * _K // 96, 96)
    valn = val.reshape(_B * _N * 16)

    def gn_lin(agg_flat, bias, gnw, gnb, Wt):
        return pl.pallas_call(
            _gn_lin_body,
            grid=(_B,),
            in_specs=[batch3(_N, _C), full(1, _C), full(1, _C), full(1, _C),
                      full(_C, _C), full(_C, 128)],
            out_specs=batch3(_N, 128),
            out_shape=jax.ShapeDtypeStruct((_B, _N, 128), jnp.float32),
        )(agg_flat.reshape(_B, _N, _C), bias.reshape(1, _C),
          gnw.reshape(1, _C), gnb.reshape(1, _C), M, Wt)

    xt2 = gn_lin(agg1, b1, gn1w, gn1b, jnp.pad(W2, ((0, 0), (0, 128 - _C))))
    agg2 = _sc_aggregate(xt2.reshape(_B * _N, 128), gidx2d, valn)
    out = gn_lin(agg2, b2, gn2w, gn2b, jnp.eye(_C, 128, dtype=jnp.float32))
    return out[:, :, :_C].transpose(0, 2, 1).reshape(_B, _C, _H, _W)


def kernel(x, W1, b1, W2, b2, gn1_w, gn1_b, gn2_w, gn2_b):
    return _run(x, W1, b1, W2, b2, gn1_w, gn1_b, gn2_w, gn2_b)
